# baseline (device time: 199436 ns/iter reference)
import functools

import jax
import jax.numpy as jnp
from jax import lax
from jax.experimental import pallas as pl
from jax.experimental.pallas import tpu as pltpu

N_DEV = 8
B_LOC = 2
SQ = 512
SKV = 512
H_LOC = 8
DH = 64
D_MODEL = 768
WINDOW = 128
SCALE = 0.125
NEG = -1e9


HALF = SQ // 2
BAND = HALF + WINDOW


def _attn_contrib(x_val, kref, vref, wq_g, wo_g, g, masks):
    qall = lax.dot_general(
        x_val, wq_g, (((1,), (0,)), ((), ())),
        preferred_element_type=jnp.float32,
    )
    qall = (qall * SCALE).astype(jnp.bfloat16)
    rows = []
    for b in range(B_LOC):
        parts = []
        for h in range(H_LOC):
            kidx = b * 64 + g * H_LOC + h
            k = jnp.squeeze(kref[pl.ds(kidx, 1)], 0)
            v = jnp.squeeze(vref[pl.ds(kidx, 1)], 0)
            halves = []
            for hf in range(2):
                q = qall[b * SQ + hf * HALF:b * SQ + (hf + 1) * HALF,
                         h * DH:(h + 1) * DH]
                ks = hf * (SQ - BAND)
                sc = lax.dot_general(
                    q, k[ks:ks + BAND], (((1,), (1,)), ((), ())),
                    preferred_element_type=jnp.float32,
                )
                w = jnp.exp(sc) * masks[hf]
                den = jnp.sum(w, axis=1, keepdims=True)
                ctx = lax.dot_general(
                    w.astype(jnp.bfloat16), v[ks:ks + BAND],
                    (((1,), (0,)), ((), ())),
                    preferred_element_type=jnp.float32,
                )
                halves.append((ctx / den).astype(jnp.bfloat16))
            parts.append(jnp.concatenate(halves, axis=0))
        rows.append(jnp.concatenate(parts, axis=1))
    ctx_all = jnp.concatenate(rows, axis=0)
    return lax.dot_general(
        ctx_all, wo_g, (((1,), (0,)), ((), ())),
        preferred_element_type=jnp.float32,
    )


def _body(x_ref, wq_ref, wo_ref, k_ref, v_ref, out_ref,
          wq_bufs, wo_bufs,
          wq_send, wq_recv, wo_send, wo_recv):
    my = lax.axis_index("i")
    right = jnp.mod(my + 1, N_DEV)
    left = jnp.mod(my + N_DEV - 1, N_DEV)

    wq_bufs[pl.ds(my, 1)] = jnp.expand_dims(wq_ref[:, :], 0)
    wo_bufs[pl.ds(my, 1)] = jnp.expand_dims(wo_ref[:, :], 0)

    barrier_sem = pltpu.get_barrier_semaphore()
    for nbr in (left, right):
        pl.semaphore_signal(
            barrier_sem, inc=1,
            device_id=(nbr,), device_id_type=pl.DeviceIdType.MESH,
        )
    pl.semaphore_wait(barrier_sem, 2)

    x_val = x_ref[:, :]
    qi = lax.broadcasted_iota(jnp.int32, (HALF, BAND), 0)
    kj = lax.broadcasted_iota(jnp.int32, (HALF, BAND), 1)
    masks = tuple(
        (jnp.abs(qi + hf * HALF - (kj + hf * (SQ - BAND))) <= WINDOW
         ).astype(jnp.float32)
        for hf in range(2)
    )

    for s in range(N_DEV):
        g = jnp.mod(my + N_DEV - s, N_DEV)
        if s < N_DEV - 1:
            rdma_wq = pltpu.make_async_remote_copy(
                src_ref=wq_bufs.at[g], dst_ref=wq_bufs.at[g],
                send_sem=wq_send.at[s], recv_sem=wq_recv.at[s],
                device_id=(right,), device_id_type=pl.DeviceIdType.MESH,
            )
            rdma_wo = pltpu.make_async_remote_copy(
                src_ref=wo_bufs.at[g], dst_ref=wo_bufs.at[g],
                send_sem=wo_send.at[s], recv_sem=wo_recv.at[s],
                device_id=(right,), device_id_type=pl.DeviceIdType.MESH,
            )
            rdma_wq.start()
            rdma_wo.start()

        wq_g = jnp.squeeze(wq_bufs[pl.ds(g, 1)], 0)
        wo_g = jnp.squeeze(wo_bufs[pl.ds(g, 1)], 0)
        contrib = _attn_contrib(x_val, k_ref, v_ref, wq_g, wo_g, g, masks)
        if s == 0:
            out_ref[:, :] = contrib
        else:
            out_ref[:, :] = out_ref[:, :] + contrib

        if s < N_DEV - 1:
            rdma_wq.wait()
            rdma_wo.wait()


def kernel(x, Wq, K_ext, V_ext, Wo):
    idx = lax.axis_index("i")

    x_l = x.astype(jnp.bfloat16).reshape(B_LOC * SQ, D_MODEL)
    wq = Wq.astype(jnp.bfloat16)
    wo = Wo.astype(jnp.bfloat16)

    k = lax.dynamic_slice_in_dim(K_ext, idx * B_LOC, B_LOC, axis=0)
    v = lax.dynamic_slice_in_dim(V_ext, idx * B_LOC, B_LOC, axis=0)
    k = jnp.transpose(k, (0, 2, 1, 3)).reshape(B_LOC * 64, SKV, DH)
    v = jnp.transpose(v, (0, 2, 1, 3)).reshape(B_LOC * 64, SKV, DH)
    k = k.astype(jnp.bfloat16)
    v = v.astype(jnp.bfloat16)

    out = pl.pallas_call(
        _body,
        out_shape=jax.ShapeDtypeStruct((B_LOC * SQ, D_MODEL), jnp.float32),
        in_specs=[
            pl.BlockSpec(memory_space=pltpu.VMEM),
            pl.BlockSpec(memory_space=pltpu.VMEM),
            pl.BlockSpec(memory_space=pltpu.VMEM),
            pl.BlockSpec(memory_space=pltpu.VMEM),
            pl.BlockSpec(memory_space=pltpu.VMEM),
        ],
        out_specs=pl.BlockSpec(memory_space=pltpu.VMEM),
        scratch_shapes=[
            pltpu.VMEM((N_DEV, D_MODEL, H_LOC * DH), jnp.bfloat16),
            pltpu.VMEM((N_DEV, H_LOC * DH, D_MODEL), jnp.bfloat16),
            pltpu.SemaphoreType.DMA((N_DEV - 1,)),
            pltpu.SemaphoreType.DMA((N_DEV - 1,)),
            pltpu.SemaphoreType.DMA((N_DEV - 1,)),
            pltpu.SemaphoreType.DMA((N_DEV - 1,)),
        ],
        compiler_params=pltpu.CompilerParams(
            collective_id=0,
            vmem_limit_bytes=100 * 1024 * 1024,
        ),
    )(x_l, wq, wo, k, v)

    return out.reshape(B_LOC, SQ, D_MODEL)


# device time: 143969 ns/iter; 1.3853x vs baseline; 1.3853x over previous
import functools

import jax
import jax.numpy as jnp
from jax import lax
from jax.experimental import pallas as pl
from jax.experimental.pallas import tpu as pltpu

N_DEV = 8
B_LOC = 2
SQ = 512
SKV = 512
H_LOC = 8
DH = 64
D_MODEL = 768
WINDOW = 128
SCALE = 0.125
NEG = -1e9


HALF = SQ // 2
BAND = HALF + WINDOW


def _attn_contrib(x_val, kref, vref, wq_g, wo_g, g, masks):
    qall = lax.dot_general(
        x_val, wq_g, (((1,), (0,)), ((), ())),
        preferred_element_type=jnp.float32,
    )
    qall = (qall * SCALE).astype(jnp.bfloat16)
    rows = []
    for b in range(B_LOC):
        parts = []
        for h in range(H_LOC):
            kidx = b * 64 + g * H_LOC + h
            k = jnp.squeeze(kref[pl.ds(kidx, 1)], 0)
            v = jnp.squeeze(vref[pl.ds(kidx, 1)], 0)
            halves = []
            for hf in range(2):
                q = qall[b * SQ + hf * HALF:b * SQ + (hf + 1) * HALF,
                         h * DH:(h + 1) * DH]
                ks = hf * (SQ - BAND)
                sc = lax.dot_general(
                    q, k[ks:ks + BAND], (((1,), (1,)), ((), ())),
                    preferred_element_type=jnp.float32,
                )
                w = jnp.exp(sc) * masks[hf]
                den = jnp.sum(w, axis=1, keepdims=True)
                ctx = lax.dot_general(
                    w.astype(jnp.bfloat16), v[ks:ks + BAND],
                    (((1,), (0,)), ((), ())),
                    preferred_element_type=jnp.float32,
                )
                halves.append((ctx / den).astype(jnp.bfloat16))
            parts.append(jnp.concatenate(halves, axis=0))
        rows.append(jnp.concatenate(parts, axis=1))
    ctx_all = jnp.concatenate(rows, axis=0)
    return lax.dot_general(
        ctx_all, wo_g, (((1,), (0,)), ((), ())),
        preferred_element_type=jnp.float32,
    )


R_STEPS = 4
L_STEPS = 3


def _body(x_ref, wq_ref, wo_ref, k_ref, v_ref, out_ref,
          wq_bufs, wo_bufs,
          wq_sr, wq_rr, wo_sr, wo_rr,
          wq_sl, wq_rl, wo_sl, wo_rl):
    my = lax.axis_index("i")
    right = jnp.mod(my + 1, N_DEV)
    left = jnp.mod(my + N_DEV - 1, N_DEV)

    wq_bufs[pl.ds(my, 1)] = jnp.expand_dims(wq_ref[:, :], 0)
    wo_bufs[pl.ds(my, 1)] = jnp.expand_dims(wo_ref[:, :], 0)

    barrier_sem = pltpu.get_barrier_semaphore()
    for nbr in (left, right):
        pl.semaphore_signal(
            barrier_sem, inc=1,
            device_id=(nbr,), device_id_type=pl.DeviceIdType.MESH,
        )
    pl.semaphore_wait(barrier_sem, 2)

    x_val = x_ref[:, :]
    qi = lax.broadcasted_iota(jnp.int32, (HALF, BAND), 0)
    kj = lax.broadcasted_iota(jnp.int32, (HALF, BAND), 1)
    masks = tuple(
        (jnp.abs(qi + hf * HALF - (kj + hf * (SQ - BAND))) <= WINDOW
         ).astype(jnp.float32)
        for hf in range(2)
    )

    def _contrib(g):
        wq_g = jnp.squeeze(wq_bufs[pl.ds(g, 1)], 0)
        wo_g = jnp.squeeze(wo_bufs[pl.ds(g, 1)], 0)
        return _attn_contrib(x_val, k_ref, v_ref, wq_g, wo_g, g, masks)

    for s in range(R_STEPS + 1):
        rdmas = []
        if s < R_STEPS:
            g_r = jnp.mod(my + N_DEV - s, N_DEV)
            rdmas.append(pltpu.make_async_remote_copy(
                src_ref=wq_bufs.at[g_r], dst_ref=wq_bufs.at[g_r],
                send_sem=wq_sr.at[s], recv_sem=wq_rr.at[s],
                device_id=(right,), device_id_type=pl.DeviceIdType.MESH,
            ))
            rdmas.append(pltpu.make_async_remote_copy(
                src_ref=wo_bufs.at[g_r], dst_ref=wo_bufs.at[g_r],
                send_sem=wo_sr.at[s], recv_sem=wo_rr.at[s],
                device_id=(right,), device_id_type=pl.DeviceIdType.MESH,
            ))
        if s < L_STEPS:
            g_l = jnp.mod(my + s, N_DEV)
            rdmas.append(pltpu.make_async_remote_copy(
                src_ref=wq_bufs.at[g_l], dst_ref=wq_bufs.at[g_l],
                send_sem=wq_sl.at[s], recv_sem=wq_rl.at[s],
                device_id=(left,), device_id_type=pl.DeviceIdType.MESH,
            ))
            rdmas.append(pltpu.make_async_remote_copy(
                src_ref=wo_bufs.at[g_l], dst_ref=wo_bufs.at[g_l],
                send_sem=wo_sl.at[s], recv_sem=wo_rl.at[s],
                device_id=(left,), device_id_type=pl.DeviceIdType.MESH,
            ))
        for r in rdmas:
            r.start()

        if s == 0:
            out_ref[:, :] = _contrib(my)
        else:
            acc = _contrib(jnp.mod(my + N_DEV - s, N_DEV))
            if s <= L_STEPS:
                acc = acc + _contrib(jnp.mod(my + s, N_DEV))
            out_ref[:, :] = out_ref[:, :] + acc

        for r in rdmas:
            r.wait()


def kernel(x, Wq, K_ext, V_ext, Wo):
    idx = lax.axis_index("i")

    x_l = x.astype(jnp.bfloat16).reshape(B_LOC * SQ, D_MODEL)
    wq = Wq.astype(jnp.bfloat16)
    wo = Wo.astype(jnp.bfloat16)

    k = lax.dynamic_slice_in_dim(K_ext, idx * B_LOC, B_LOC, axis=0)
    v = lax.dynamic_slice_in_dim(V_ext, idx * B_LOC, B_LOC, axis=0)
    k = jnp.transpose(k, (0, 2, 1, 3)).reshape(B_LOC * 64, SKV, DH)
    v = jnp.transpose(v, (0, 2, 1, 3)).reshape(B_LOC * 64, SKV, DH)
    k = k.astype(jnp.bfloat16)
    v = v.astype(jnp.bfloat16)

    out = pl.pallas_call(
        _body,
        out_shape=jax.ShapeDtypeStruct((B_LOC * SQ, D_MODEL), jnp.float32),
        in_specs=[
            pl.BlockSpec(memory_space=pltpu.VMEM),
            pl.BlockSpec(memory_space=pltpu.VMEM),
            pl.BlockSpec(memory_space=pltpu.VMEM),
            pl.BlockSpec(memory_space=pltpu.VMEM),
            pl.BlockSpec(memory_space=pltpu.VMEM),
        ],
        out_specs=pl.BlockSpec(memory_space=pltpu.VMEM),
        scratch_shapes=[
            pltpu.VMEM((N_DEV, D_MODEL, H_LOC * DH), jnp.bfloat16),
            pltpu.VMEM((N_DEV, H_LOC * DH, D_MODEL), jnp.bfloat16),
            pltpu.SemaphoreType.DMA((R_STEPS,)),
            pltpu.SemaphoreType.DMA((R_STEPS,)),
            pltpu.SemaphoreType.DMA((R_STEPS,)),
            pltpu.SemaphoreType.DMA((R_STEPS,)),
            pltpu.SemaphoreType.DMA((L_STEPS,)),
            pltpu.SemaphoreType.DMA((L_STEPS,)),
            pltpu.SemaphoreType.DMA((L_STEPS,)),
            pltpu.SemaphoreType.DMA((L_STEPS,)),
        ],
        compiler_params=pltpu.CompilerParams(
            collective_id=0,
            vmem_limit_bytes=100 * 1024 * 1024,
        ),
    )(x_l, wq, wo, k, v)

    return out.reshape(B_LOC, SQ, D_MODEL)
